# baseline (speedup 1.0000x reference)
import functools

import jax
import jax.numpy as jnp
from jax import lax
from jax.experimental import pallas as pl
from jax.experimental.pallas import tpu as pltpu
from jax.experimental.pallas import tpu_sc as plsc

LANES = 16
SUB = 8


@functools.lru_cache(maxsize=None)
def _build(V, D, B, L):
    info = plsc.get_sparse_core_info()
    NC, NS = info.num_cores, info.num_subcores
    NW = NC * NS
    assert B % NW == 0 and D % LANES == 0 and V % SUB == 0
    b_per_w = B // NW
    n_d = D // LANES
    CC = 4
    pairs = CC * L
    assert pairs % LANES == 0
    n_batch = pairs // LANES
    n_chunks = b_per_w // CC
    CEN_G = 64
    n_cen_g = b_per_w // CEN_G

    mesh = plsc.VectorSubcoreMesh(core_axis_name="c", subcore_axis_name="s")
    params = pltpu.CompilerParams(use_tc_tiling_on_sc=True)

    @functools.partial(
        pl.kernel,
        mesh=mesh,
        out_type=jax.ShapeDtypeStruct((B, D), jnp.float32),
        scratch_types=[
            pltpu.VMEM((b_per_w,), jnp.int32),
            pltpu.VMEM((b_per_w, D), jnp.float32),
            pltpu.SemaphoreType.DMA,
        ],
        compiler_params=params,
    )
    def cen_kernel(cen_tab, cen_idx, out_hbm, cen_idx_v, cen_rows_v, sem):
        wid = lax.axis_index("s") * NC + lax.axis_index("c")
        base = wid * b_per_w
        pltpu.sync_copy(cen_idx.at[pl.ds(base, b_per_w)], cen_idx_v)

        def cen_fetch_body(g, _):
            vecs = [cen_idx_v[pl.ds(g * CEN_G + k * LANES, LANES)]
                    for k in range(CEN_G // LANES)]
            cps = []
            for i in range(CEN_G):
                b = g * CEN_G + i
                w = vecs[i // LANES][i % LANES]
                cps.append(pltpu.async_copy(
                    cen_tab.at[w >> 1, pl.ds((w & 1) * D, D)],
                    cen_rows_v.at[b], sem))
            for cp in cps:
                cp.wait()
            return 0
        lax.fori_loop(0, n_cen_g, cen_fetch_body, 0)
        pltpu.sync_copy(cen_rows_v, out_hbm.at[pl.ds(base, b_per_w)])

    @functools.partial(
        pl.kernel,
        mesh=mesh,
        out_type=jax.ShapeDtypeStruct((B * L,), jnp.float32),
        scratch_types=[
            pltpu.VMEM((pairs,), jnp.int32),
            pltpu.VMEM((b_per_w, D), jnp.float32),
            pltpu.VMEM((pairs, D), jnp.float32),
            pltpu.VMEM((LANES, 2 * LANES), jnp.float32),
            pltpu.VMEM((2 * LANES,), jnp.float32),
            pltpu.VMEM((b_per_w * L,), jnp.float32),
            [pltpu.SemaphoreType.DMA] * n_batch,
        ],
        compiler_params=params,
    )
    def ctx_kernel(ctx_tab, ctx_idx, cen_rows, out_hbm,
                   ctx_idx_v, cen_rows_v, ctx_rows_v, rbuf_v, sbuf_v,
                   scores_v, sems):
        wid = lax.axis_index("s") * NC + lax.axis_index("c")
        base = wid * b_per_w
        zeros16 = jnp.zeros((LANES,), jnp.float32)
        for e in range(LANES):
            rbuf_v[e, pl.ds(LANES, LANES)] = zeros16
        pltpu.sync_copy(cen_rows.at[pl.ds(base, b_per_w)], cen_rows_v)

        def chunk_body(c, _):
            pltpu.sync_copy(
                ctx_idx.at[pl.ds(base * L + c * pairs, pairs)], ctx_idx_v)
            ivecs = [ctx_idx_v[pl.ds(k * LANES, LANES)]
                     for k in range(pairs // LANES)]
            cps = [[] for _ in range(n_batch)]
            for r in range(pairs):
                w = ivecs[r // LANES][r % LANES]
                cps[r // LANES].append(pltpu.async_copy(
                    ctx_tab.at[w >> 1, pl.ds((w & 1) * D, D)],
                    ctx_rows_v.at[r], sems[r // LANES]))

            cen = [[cen_rows_v[c * CC + j, pl.ds(t * LANES, LANES)]
                    for t in range(n_d)] for j in range(CC)]
            for k in range(n_batch):
                for cp in cps[k]:
                    cp.wait()
                for e in range(LANES):
                    r = k * LANES + e
                    cj = cen[r // L]
                    acc = ctx_rows_v[r, pl.ds(0, LANES)] * cj[0]
                    for t in range(1, n_d):
                        acc += ctx_rows_v[r, pl.ds(t * LANES, LANES)] * cj[t]
                    t_ = acc
                    for off in (SUB, 4, 2, 1):
                        rbuf_v[e, pl.ds(0, LANES)] = t_
                        t_ = t_ + rbuf_v[e, pl.ds(off, LANES)]
                    sbuf_v[pl.ds(e, LANES)] = t_
                scores_v[pl.ds(c * pairs + k * LANES, LANES)] = (
                    sbuf_v[pl.ds(0, LANES)])
            return 0
        lax.fori_loop(0, n_chunks, chunk_body, 0)

        pltpu.sync_copy(scores_v, out_hbm.at[pl.ds(base * L, b_per_w * L)])

    return cen_kernel, ctx_kernel


def kernel(center_word, context_words, center_table, context_table):
    V, D = center_table.shape
    B, L = context_words.shape
    cen_idx = center_word.astype(jnp.int32)
    ctx_idx = context_words.reshape(-1).astype(jnp.int32)
    cen2 = center_table.reshape(V // 2, 2 * D)
    ctx2 = context_table.reshape(V // 2, 2 * D)
    cen_kernel, ctx_kernel = _build(V, D, B, L)
    cen_rows = cen_kernel(cen2, cen_idx)
    scores = ctx_kernel(ctx2, ctx_idx, cen_rows)
    return scores.reshape(B, L)


# speedup vs baseline: 1.5596x; 1.5596x over previous
import functools

import jax
import jax.numpy as jnp
from jax import lax
from jax.experimental import pallas as pl
from jax.experimental.pallas import tpu as pltpu
from jax.experimental.pallas import tpu_sc as plsc

LANES = 16
SUB = 8


@functools.lru_cache(maxsize=None)
def _build(V, D, B, L):
    info = plsc.get_sparse_core_info()
    NC, NS = info.num_cores, info.num_subcores
    NW = NC * NS
    assert B % NW == 0 and D % LANES == 0 and V % SUB == 0
    b_per_w = B // NW
    n_d = D // LANES
    CC = 8
    pairs = CC * L
    assert pairs % LANES == 0
    n_batch = pairs // LANES
    n_chunks = b_per_w // CC
    CEN_G = 64
    n_cen_g = b_per_w // CEN_G

    mesh = plsc.VectorSubcoreMesh(core_axis_name="c", subcore_axis_name="s")
    params = pltpu.CompilerParams(use_tc_tiling_on_sc=True)

    @functools.partial(
        pl.kernel,
        mesh=mesh,
        out_type=jax.ShapeDtypeStruct((B, D), jnp.float32),
        scratch_types=[
            pltpu.VMEM((b_per_w,), jnp.int32),
            pltpu.VMEM((b_per_w, D), jnp.float32),
            pltpu.SemaphoreType.DMA,
        ],
        compiler_params=params,
    )
    def cen_kernel(cen_tab, cen_idx, out_hbm, cen_idx_v, cen_rows_v, sem):
        wid = lax.axis_index("s") * NC + lax.axis_index("c")
        base = wid * b_per_w
        pltpu.sync_copy(cen_idx.at[pl.ds(base, b_per_w)], cen_idx_v)

        def cen_fetch_body(g, _):
            vecs = [cen_idx_v[pl.ds(g * CEN_G + k * LANES, LANES)]
                    for k in range(CEN_G // LANES)]
            cps = []
            for i in range(CEN_G):
                b = g * CEN_G + i
                w = vecs[i // LANES][i % LANES]
                cps.append(pltpu.async_copy(
                    cen_tab.at[w >> 3, w & 7], cen_rows_v.at[b], sem))
            for cp in cps:
                cp.wait()
            return 0
        lax.fori_loop(0, n_cen_g, cen_fetch_body, 0)
        pltpu.sync_copy(cen_rows_v, out_hbm.at[pl.ds(base, b_per_w)])

    @functools.partial(
        pl.kernel,
        mesh=mesh,
        out_type=jax.ShapeDtypeStruct((B * L,), jnp.float32),
        scratch_types=[
            pltpu.VMEM((b_per_w * L,), jnp.int32),
            pltpu.VMEM((b_per_w, D), jnp.float32),
            pltpu.VMEM((pairs, D), jnp.float32),
            pltpu.VMEM((LANES, 2 * LANES), jnp.float32),
            pltpu.VMEM((2 * LANES,), jnp.float32),
            pltpu.VMEM((b_per_w * L,), jnp.float32),
            [pltpu.SemaphoreType.DMA] * n_batch,
        ],
        compiler_params=params,
    )
    def ctx_kernel(ctx_tab, ctx_idx, cen_rows, out_hbm,
                   ctx_idx_v, cen_rows_v, ctx_rows_v, rbuf_v, sbuf_v,
                   scores_v, sems):
        wid = lax.axis_index("s") * NC + lax.axis_index("c")
        base = wid * b_per_w
        zeros16 = jnp.zeros((LANES,), jnp.float32)
        for e in range(LANES):
            rbuf_v[e, pl.ds(LANES, LANES)] = zeros16
        pltpu.sync_copy(ctx_idx.at[pl.ds(base * L, b_per_w * L)], ctx_idx_v)
        pltpu.sync_copy(cen_rows.at[pl.ds(base, b_per_w)], cen_rows_v)

        def chunk_body(c, _):
            ivecs = [ctx_idx_v[pl.ds(c * pairs + k * LANES, LANES)]
                     for k in range(pairs // LANES)]
            cps = [[] for _ in range(n_batch)]
            for r in range(pairs):
                w = ivecs[r // LANES][r % LANES]
                cps[r // LANES].append(pltpu.async_copy(
                    ctx_tab.at[w], ctx_rows_v.at[r],
                    sems[r // LANES]))

            cen = [[cen_rows_v[c * CC + j, pl.ds(t * LANES, LANES)]
                    for t in range(n_d)] for j in range(CC)]
            for k in range(n_batch):
                for cp in cps[k]:
                    cp.wait()
                for e in range(LANES):
                    r = k * LANES + e
                    cj = cen[r // L]
                    acc = ctx_rows_v[r, pl.ds(0, LANES)] * cj[0]
                    for t in range(1, n_d):
                        acc += ctx_rows_v[r, pl.ds(t * LANES, LANES)] * cj[t]
                    t_ = acc
                    for off in (SUB, 4, 2, 1):
                        rbuf_v[e, pl.ds(0, LANES)] = t_
                        t_ = t_ + rbuf_v[e, pl.ds(off, LANES)]
                    sbuf_v[pl.ds(e, LANES)] = t_
                scores_v[pl.ds(c * pairs + k * LANES, LANES)] = (
                    sbuf_v[pl.ds(0, LANES)])
            return 0
        lax.fori_loop(0, n_chunks, chunk_body, 0)

        pltpu.sync_copy(scores_v, out_hbm.at[pl.ds(base * L, b_per_w * L)])

    return cen_kernel, ctx_kernel


def kernel(center_word, context_words, center_table, context_table):
    V, D = center_table.shape
    B, L = context_words.shape
    cen_idx = center_word.astype(jnp.int32)
    ctx_idx = context_words.reshape(-1).astype(jnp.int32)
    cen3 = center_table.reshape(V // SUB, SUB, D)
    cen_kernel, ctx_kernel = _build(V, D, B, L)
    cen_rows = cen_kernel(cen3, cen_idx)
    scores = ctx_kernel(context_table, ctx_idx, cen_rows)
    return scores.reshape(B, L)


# speedup vs baseline: 1.9031x; 1.2202x over previous
import functools

import jax
import jax.numpy as jnp
from jax import lax
from jax.experimental import pallas as pl
from jax.experimental.pallas import tpu as pltpu
from jax.experimental.pallas import tpu_sc as plsc

LANES = 16
SUB = 8


@functools.lru_cache(maxsize=None)
def _build(V, D, B, L):
    info = plsc.get_sparse_core_info()
    NC, NS = info.num_cores, info.num_subcores
    NW = NC * NS
    assert B % NW == 0 and D % LANES == 0 and V % SUB == 0
    b_per_w = B // NW
    n_d = D // LANES
    CC = 4
    pairs = CC * L
    assert pairs % LANES == 0
    n_batch = pairs // LANES
    n_chunks = b_per_w // CC
    CEN_G = 64
    n_cen_g = b_per_w // CEN_G

    mesh = plsc.VectorSubcoreMesh(core_axis_name="c", subcore_axis_name="s")
    params = pltpu.CompilerParams(use_tc_tiling_on_sc=True)

    @functools.partial(
        pl.kernel,
        mesh=mesh,
        out_type=jax.ShapeDtypeStruct((B, D), jnp.float32),
        scratch_types=[
            pltpu.VMEM((b_per_w,), jnp.int32),
            pltpu.VMEM((b_per_w, D), jnp.float32),
            pltpu.SemaphoreType.DMA,
        ],
        compiler_params=params,
    )
    def cen_kernel(cen_tab, cen_idx, out_hbm, cen_idx_v, cen_rows_v, sem):
        wid = lax.axis_index("s") * NC + lax.axis_index("c")
        base = wid * b_per_w
        pltpu.sync_copy(cen_idx.at[pl.ds(base, b_per_w)], cen_idx_v)

        def cen_fetch_body(g, _):
            vecs = [cen_idx_v[pl.ds(g * CEN_G + k * LANES, LANES)]
                    for k in range(CEN_G // LANES)]
            cps = []
            for i in range(CEN_G):
                b = g * CEN_G + i
                w = vecs[i // LANES][i % LANES]
                cps.append(pltpu.async_copy(
                    cen_tab.at[w], cen_rows_v.at[b], sem))
            for cp in cps:
                cp.wait()
            return 0
        lax.fori_loop(0, n_cen_g, cen_fetch_body, 0)
        pltpu.sync_copy(cen_rows_v, out_hbm.at[pl.ds(base, b_per_w)])

    @functools.partial(
        pl.kernel,
        mesh=mesh,
        out_type=jax.ShapeDtypeStruct((B * L,), jnp.float32),
        scratch_types=[
            pltpu.VMEM((b_per_w * L,), jnp.int32),
            pltpu.VMEM((b_per_w, D), jnp.float32),
            pltpu.VMEM((pairs, D), jnp.float32),
            pltpu.VMEM((LANES, 2 * LANES), jnp.float32),
            pltpu.VMEM((2 * LANES,), jnp.float32),
            pltpu.VMEM((b_per_w * L,), jnp.float32),
            [pltpu.SemaphoreType.DMA] * n_batch,
        ],
        compiler_params=params,
    )
    def ctx_kernel(ctx_tab, ctx_idx, cen_rows, out_hbm,
                   ctx_idx_v, cen_rows_v, ctx_rows_v, rbuf_v, sbuf_v,
                   scores_v, sems):
        wid = lax.axis_index("s") * NC + lax.axis_index("c")
        base = wid * b_per_w
        zeros16 = jnp.zeros((LANES,), jnp.float32)
        for e in range(LANES):
            rbuf_v[e, pl.ds(LANES, LANES)] = zeros16
        pltpu.sync_copy(ctx_idx.at[pl.ds(base * L, b_per_w * L)], ctx_idx_v)
        pltpu.sync_copy(cen_rows.at[pl.ds(base, b_per_w)], cen_rows_v)

        def chunk_body(c, _):
            ivecs = [ctx_idx_v[pl.ds(c * pairs + k * LANES, LANES)]
                     for k in range(pairs // LANES)]
            cps = [[] for _ in range(n_batch)]
            for r in range(pairs):
                w = ivecs[r // LANES][r % LANES]
                cps[r // LANES].append(pltpu.async_copy(
                    ctx_tab.at[w >> 3, w & 7], ctx_rows_v.at[r],
                    sems[r // LANES]))

            cen = [[cen_rows_v[c * CC + j, pl.ds(t * LANES, LANES)]
                    for t in range(n_d)] for j in range(CC)]
            for k in range(n_batch):
                for cp in cps[k]:
                    cp.wait()
                for e in range(LANES):
                    r = k * LANES + e
                    cj = cen[r // L]
                    acc = ctx_rows_v[r, pl.ds(0, LANES)] * cj[0]
                    for t in range(1, n_d):
                        acc += ctx_rows_v[r, pl.ds(t * LANES, LANES)] * cj[t]
                    t_ = acc
                    for off in (SUB, 4, 2, 1):
                        rbuf_v[e, pl.ds(0, LANES)] = t_
                        t_ = t_ + rbuf_v[e, pl.ds(off, LANES)]
                    sbuf_v[pl.ds(e, LANES)] = t_
                scores_v[pl.ds(c * pairs + k * LANES, LANES)] = (
                    sbuf_v[pl.ds(0, LANES)])
            return 0
        lax.fori_loop(0, n_chunks, chunk_body, 0)

        pltpu.sync_copy(scores_v, out_hbm.at[pl.ds(base * L, b_per_w * L)])

    return cen_kernel, ctx_kernel


def kernel(center_word, context_words, center_table, context_table):
    V, D = center_table.shape
    B, L = context_words.shape
    cen_idx = center_word.astype(jnp.int32)
    ctx_idx = context_words.reshape(-1).astype(jnp.int32)
    ctx3 = context_table.reshape(V // SUB, SUB, D)
    cen_kernel, ctx_kernel = _build(V, D, B, L)
    cen_rows = cen_kernel(center_table, cen_idx)
    scores = ctx_kernel(ctx3, ctx_idx, cen_rows)
    return scores.reshape(B, L)


# speedup vs baseline: 2.0166x; 1.0597x over previous
import functools

import jax
import jax.numpy as jnp
from jax import lax
from jax.experimental import pallas as pl
from jax.experimental.pallas import tpu as pltpu
from jax.experimental.pallas import tpu_sc as plsc

LANES = 16
SUB = 8


@functools.lru_cache(maxsize=None)
def _build(V, D, B, L):
    info = plsc.get_sparse_core_info()
    NC, NS = info.num_cores, info.num_subcores
    NW = NC * NS
    assert B % NW == 0 and D % LANES == 0 and V % SUB == 0
    b_per_w = B // NW
    n_d = D // LANES
    CC = 4
    pairs = CC * L
    assert pairs % LANES == 0
    n_batch = pairs // LANES
    n_chunks = b_per_w // CC
    CEN_G = 64
    n_cen_g = b_per_w // CEN_G

    mesh = plsc.VectorSubcoreMesh(core_axis_name="c", subcore_axis_name="s")
    params = pltpu.CompilerParams(use_tc_tiling_on_sc=True)

    @functools.partial(
        pl.kernel,
        mesh=mesh,
        out_type=jax.ShapeDtypeStruct((B, D), jnp.float32),
        scratch_types=[
            pltpu.VMEM((b_per_w,), jnp.int32),
            pltpu.VMEM((b_per_w, D), jnp.float32),
            pltpu.SemaphoreType.DMA,
        ],
        compiler_params=params,
    )
    def cen_kernel(cen_tab, cen_idx, out_hbm, cen_idx_v, cen_rows_v, sem):
        wid = lax.axis_index("s") * NC + lax.axis_index("c")
        base = wid * b_per_w
        pltpu.sync_copy(cen_idx.at[pl.ds(base, b_per_w)], cen_idx_v)

        def cen_fetch_body(g, _):
            vecs = [cen_idx_v[pl.ds(g * CEN_G + k * LANES, LANES)]
                    for k in range(CEN_G // LANES)]
            cps = []
            for i in range(CEN_G):
                b = g * CEN_G + i
                w = vecs[i // LANES][i % LANES]
                cps.append(pltpu.async_copy(
                    cen_tab.at[w >> 3, w & 7], cen_rows_v.at[b], sem))
            for cp in cps:
                cp.wait()
            return 0
        lax.fori_loop(0, n_cen_g, cen_fetch_body, 0)
        pltpu.sync_copy(cen_rows_v, out_hbm.at[pl.ds(base, b_per_w)])

    @functools.partial(
        pl.kernel,
        mesh=mesh,
        out_type=jax.ShapeDtypeStruct((B * L,), jnp.float32),
        scratch_types=[
            pltpu.VMEM((b_per_w * L,), jnp.int32),
            pltpu.VMEM((b_per_w, D), jnp.float32),
            pltpu.VMEM((pairs, D), jnp.float32),
            pltpu.VMEM((LANES, 2 * LANES), jnp.float32),
            pltpu.VMEM((2 * LANES,), jnp.float32),
            pltpu.VMEM((b_per_w * L,), jnp.float32),
            [pltpu.SemaphoreType.DMA] * n_batch,
        ],
        compiler_params=params,
    )
    def ctx_kernel(ctx_tab, ctx_idx, cen_rows, out_hbm,
                   ctx_idx_v, cen_rows_v, ctx_rows_v, rbuf_v, sbuf_v,
                   scores_v, sems):
        wid = lax.axis_index("s") * NC + lax.axis_index("c")
        base = wid * b_per_w
        zeros16 = jnp.zeros((LANES,), jnp.float32)
        for e in range(LANES):
            rbuf_v[e, pl.ds(LANES, LANES)] = zeros16
        pltpu.sync_copy(ctx_idx.at[pl.ds(base * L, b_per_w * L)], ctx_idx_v)
        pltpu.sync_copy(cen_rows.at[pl.ds(base, b_per_w)], cen_rows_v)

        def chunk_body(c, _):
            ivecs = [ctx_idx_v[pl.ds(c * pairs + k * LANES, LANES)]
                     for k in range(pairs // LANES)]
            cps = [[] for _ in range(n_batch)]
            for r in range(pairs):
                w = ivecs[r // LANES][r % LANES]
                cps[r // LANES].append(pltpu.async_copy(
                    ctx_tab.at[w >> 3, w & 7], ctx_rows_v.at[r],
                    sems[r // LANES]))

            cen = [[cen_rows_v[c * CC + j, pl.ds(t * LANES, LANES)]
                    for t in range(n_d)] for j in range(CC)]
            for k in range(n_batch):
                for cp in cps[k]:
                    cp.wait()
                for e in range(LANES):
                    r = k * LANES + e
                    cj = cen[r // L]
                    acc = ctx_rows_v[r, pl.ds(0, LANES)] * cj[0]
                    for t in range(1, n_d):
                        acc += ctx_rows_v[r, pl.ds(t * LANES, LANES)] * cj[t]
                    t_ = acc
                    for off in (SUB, 4, 2, 1):
                        rbuf_v[e, pl.ds(0, LANES)] = t_
                        t_ = t_ + rbuf_v[e, pl.ds(off, LANES)]
                    sbuf_v[pl.ds(e, LANES)] = t_
                scores_v[pl.ds(c * pairs + k * LANES, LANES)] = (
                    sbuf_v[pl.ds(0, LANES)])
            return 0
        lax.fori_loop(0, n_chunks, chunk_body, 0)

        pltpu.sync_copy(scores_v, out_hbm.at[pl.ds(base * L, b_per_w * L)])

    return cen_kernel, ctx_kernel


def kernel(center_word, context_words, center_table, context_table):
    V, D = center_table.shape
    B, L = context_words.shape
    cen_idx = center_word.astype(jnp.int32)
    ctx_idx = context_words.reshape(-1).astype(jnp.int32)
    ctx3 = context_table.reshape(V // SUB, SUB, D)
    cen3 = center_table.reshape(V // SUB, SUB, D)
    cen_kernel, ctx_kernel = _build(V, D, B, L)
    cen_rows = cen_kernel(cen3, cen_idx)
    scores = ctx_kernel(ctx3, ctx_idx, cen_rows)
    return scores.reshape(B, L)


# speedup vs baseline: 2.0421x; 1.0126x over previous
import functools

import jax
import jax.numpy as jnp
from jax import lax
from jax.experimental import pallas as pl
from jax.experimental.pallas import tpu as pltpu
from jax.experimental.pallas import tpu_sc as plsc

LANES = 16
SUB = 8


@functools.lru_cache(maxsize=None)
def _build(V, D, B, L):
    info = plsc.get_sparse_core_info()
    NC, NS = info.num_cores, info.num_subcores
    NW = NC * NS
    assert B % NW == 0 and D % LANES == 0 and V % SUB == 0
    b_per_w = B // NW
    n_d = D // LANES
    CC = 4
    pairs = CC * L
    assert pairs % LANES == 0
    n_batch = pairs // LANES
    n_chunks = b_per_w // CC
    CEN_G = 64
    n_cen_g = b_per_w // CEN_G

    mesh = plsc.VectorSubcoreMesh(core_axis_name="c", subcore_axis_name="s")
    params = pltpu.CompilerParams(use_tc_tiling_on_sc=True)

    @functools.partial(
        pl.kernel,
        mesh=mesh,
        out_type=jax.ShapeDtypeStruct((B * L,), jnp.float32),
        scratch_types=[
            pltpu.VMEM((b_per_w,), jnp.int32),
            pltpu.VMEM((b_per_w * L,), jnp.int32),
            pltpu.VMEM((b_per_w, D), jnp.float32),
            pltpu.VMEM((pairs, D), jnp.float32),
            pltpu.VMEM((LANES, 2 * LANES), jnp.float32),
            pltpu.VMEM((2 * LANES,), jnp.float32),
            pltpu.VMEM((b_per_w * L,), jnp.float32),
            [pltpu.SemaphoreType.DMA] * n_batch,
        ],
        compiler_params=params,
    )
    def w2v_kernel(cen_tab, ctx_tab, cen_idx, ctx_idx, out_hbm,
                   cen_idx_v, ctx_idx_v, cen_rows_v, ctx_rows_v, rbuf_v,
                   sbuf_v, scores_v, sems):
        wid = lax.axis_index("s") * NC + lax.axis_index("c")
        base = wid * b_per_w
        zeros16 = jnp.zeros((LANES,), jnp.float32)
        for e in range(LANES):
            rbuf_v[e, pl.ds(LANES, LANES)] = zeros16
        pltpu.sync_copy(ctx_idx.at[pl.ds(base * L, b_per_w * L)], ctx_idx_v)
        pltpu.sync_copy(cen_idx.at[pl.ds(base, b_per_w)], cen_idx_v)

        def cen_fetch_body(g, _):
            vecs = [cen_idx_v[pl.ds(g * CEN_G + k * LANES, LANES)]
                    for k in range(CEN_G // LANES)]
            cps = []
            for i in range(CEN_G):
                b = g * CEN_G + i
                w = vecs[i // LANES][i % LANES]
                cps.append(pltpu.async_copy(
                    cen_tab.at[w >> 3, w & 7], cen_rows_v.at[b],
                    sems[0]))
            for cp in cps:
                cp.wait()
            return 0
        lax.fori_loop(0, n_cen_g, cen_fetch_body, 0)

        def chunk_body(c, _):
            ivecs = [ctx_idx_v[pl.ds(c * pairs + k * LANES, LANES)]
                     for k in range(pairs // LANES)]
            cps = [[] for _ in range(n_batch)]
            for r in range(pairs):
                w = ivecs[r // LANES][r % LANES]
                cps[r // LANES].append(pltpu.async_copy(
                    ctx_tab.at[w >> 3, w & 7], ctx_rows_v.at[r],
                    sems[r // LANES]))

            cen = [[cen_rows_v[c * CC + j, pl.ds(t * LANES, LANES)]
                    for t in range(n_d)] for j in range(CC)]
            for k in range(n_batch):
                for cp in cps[k]:
                    cp.wait()
                for e in range(LANES):
                    r = k * LANES + e
                    cj = cen[r // L]
                    acc = ctx_rows_v[r, pl.ds(0, LANES)] * cj[0]
                    for t in range(1, n_d):
                        acc += ctx_rows_v[r, pl.ds(t * LANES, LANES)] * cj[t]
                    t_ = acc
                    for off in (SUB, 4, 2, 1):
                        rbuf_v[e, pl.ds(0, LANES)] = t_
                        t_ = t_ + rbuf_v[e, pl.ds(off, LANES)]
                    sbuf_v[pl.ds(e, LANES)] = t_
                scores_v[pl.ds(c * pairs + k * LANES, LANES)] = (
                    sbuf_v[pl.ds(0, LANES)])
            return 0
        lax.fori_loop(0, n_chunks, chunk_body, 0)

        pltpu.sync_copy(scores_v, out_hbm.at[pl.ds(base * L, b_per_w * L)])

    return w2v_kernel


def kernel(center_word, context_words, center_table, context_table):
    V, D = center_table.shape
    B, L = context_words.shape
    cen_idx = center_word.astype(jnp.int32)
    ctx_idx = context_words.reshape(-1).astype(jnp.int32)
    ctx3 = context_table.reshape(V // SUB, SUB, D)
    cen3 = center_table.reshape(V // SUB, SUB, D)
    w2v_kernel = _build(V, D, B, L)
    scores = w2v_kernel(cen3, ctx3, cen_idx, ctx_idx)
    return scores.reshape(B, L)
